# Initial kernel scaffold; baseline (speedup 1.0000x reference)
import jax
import jax.numpy as jnp
from jax.experimental import pallas as pl


def kernel(x, edge_index, W_lin, b_lin, W1, b1, W2, b2):
    raise NotImplementedError("write your pallas kernel here")


# speedup vs baseline: 7.3777x; 7.3777x over previous
import functools

import jax
import jax.numpy as jnp
from jax import lax
from jax.experimental import pallas as pl
from jax.experimental.pallas import tpu as pltpu
from jax.experimental.pallas import tpu_sc as plsc

D = 128
DW = 144
NC, NS = 2, 16
CH = 80


def _sc_scatter(xaug, src, dst, zrows):
    n = xaug.shape[0]
    e = src.shape[0]
    rps = (n // NS) // 8 * 8
    tail = n - (NS - 1) * rps
    chunks = e // (NC * NS * CH)
    epw = chunks * CH
    mesh = plsc.VectorSubcoreMesh(core_axis_name="c", subcore_axis_name="s",
                                  num_cores=NC, num_subcores=NS)

    @functools.partial(
        pl.kernel,
        out_type=jax.ShapeDtypeStruct((NC * n, DW), jnp.float32),
        mesh=mesh,
        scratch_types=[
            pltpu.VMEM((CH,), jnp.int32),
            pltpu.VMEM((CH,), jnp.int32),
            pltpu.VMEM((CH, DW), jnp.float32),
            pltpu.VMEM_SHARED((n, DW), jnp.float32),
            pltpu.SemaphoreType.DMA,
        ],
        compiler_params=pltpu.CompilerParams(use_tc_tiling_on_sc=False),
    )
    def k(xaug_hbm, src_hbm, dst_hbm, z_hbm, out_hbm, src_v, dst_v, rows_v,
          s_sh, sem):
        c = lax.axis_index("c")
        s = lax.axis_index("s")
        wid = c * NS + s
        @pl.when(s < NS - 1)
        def _z0():
            pltpu.sync_copy(z_hbm.at[pl.ds(0, rps)],
                            s_sh.at[pl.ds(s * rps, rps)])

        @pl.when(s == NS - 1)
        def _z1():
            pltpu.sync_copy(z_hbm, s_sh.at[pl.ds((NS - 1) * rps, tail)])

        plsc.subcore_barrier()

        def chunk(i, carry):
            base = wid * epw + i * CH
            pltpu.sync_copy(src_hbm.at[pl.ds(base, CH)], src_v)
            pltpu.sync_copy(dst_hbm.at[pl.ds(base, CH)], dst_v)
            pltpu.async_copy(xaug_hbm.at[src_v], rows_v, sem).wait()
            pltpu.sync_copy(rows_v, s_sh.at[dst_v], add=True)
            return carry

        lax.fori_loop(0, chunks, chunk, 0)
        plsc.subcore_barrier()

        @pl.when(s < NS - 1)
        def _r0():
            pltpu.sync_copy(s_sh.at[pl.ds(s * rps, rps)],
                            out_hbm.at[pl.ds(c * n + s * rps, rps)])

        @pl.when(s == NS - 1)
        def _r1():
            pltpu.sync_copy(s_sh.at[pl.ds((NS - 1) * rps, tail)],
                            out_hbm.at[pl.ds(c * n + (NS - 1) * rps, tail)])

    return k(xaug, src, dst, zrows)


def _tc_dense(sacc, x, wlt, bl, w1t, b1r, w2pt, b2p):
    n = x.shape[0]
    br = 1000
    g = n // br

    def body(s0, s1, xb, wlt_r, bl_r, w1t_r, b1_r, w2t_r, b2_r, out, acc):
        i = pl.program_id(0)

        @pl.when(i == 0)
        def _init():
            acc[...] = jnp.zeros_like(acc)

        sa = s0[...] + s1[...]
        feat = sa[:, :D] + xb[...]
        cnt = sa[:, D:D + 1] + 1.0
        agg = feat / cnt
        t = jnp.dot(agg, wlt_r[...], preferred_element_type=jnp.float32) + bl_r[...]
        h = jnp.maximum(
            jnp.dot(t, w1t_r[...], preferred_element_type=jnp.float32) + b1_r[...],
            0.0)
        acc[...] = acc[...] + jnp.sum(h, axis=0, keepdims=True)

        @pl.when(i == g - 1)
        def _fin():
            out[...] = (jnp.dot(acc[...], w2t_r[...],
                                preferred_element_type=jnp.float32)
                        + b2_r[...] * float(n))

    full = lambda i: (0, 0)
    return pl.pallas_call(
        body,
        grid=(g,),
        in_specs=[
            pl.BlockSpec((br, DW), lambda i: (i, 0)),
            pl.BlockSpec((br, DW), lambda i, _g=g: (i + _g, 0)),
            pl.BlockSpec((br, D), lambda i: (i, 0)),
            pl.BlockSpec((D, D), full),
            pl.BlockSpec((1, D), full),
            pl.BlockSpec((D, D), full),
            pl.BlockSpec((1, D), full),
            pl.BlockSpec((D, D), full),
            pl.BlockSpec((1, D), full),
        ],
        out_specs=pl.BlockSpec((1, D), full),
        out_shape=jax.ShapeDtypeStruct((1, D), jnp.float32),
        scratch_shapes=[pltpu.VMEM((1, D), jnp.float32)],
    )(sacc, sacc, x, wlt, bl, w1t, b1r, w2pt, b2p)


def kernel(x, edge_index, W_lin, b_lin, W1, b1, W2, b2):
    n, d = x.shape
    out_w = W2.shape[0]
    src = edge_index[0]
    dst = edge_index[1]
    xaug = jnp.concatenate(
        [x, jnp.ones((n, 1), jnp.float32), jnp.zeros((n, DW - d - 1), jnp.float32)],
        axis=1)
    zrows = jnp.zeros((n - (NS - 1) * ((n // NS) // 8 * 8), DW), jnp.float32)
    sacc = _sc_scatter(xaug, src, dst, zrows)
    w2pt = jnp.zeros((D, D), jnp.float32).at[:, :out_w].set(W2.T)
    b2p = jnp.zeros((1, D), jnp.float32).at[0, :out_w].set(b2)
    out_row = _tc_dense(sacc, x, W_lin.T, b_lin.reshape(1, D), W1.T,
                        b1.reshape(1, D), w2pt, b2p)
    return out_row[0, :out_w]


# speedup vs baseline: 12.0358x; 1.6314x over previous
import functools

import jax
import jax.numpy as jnp
from jax import lax
from jax.experimental import pallas as pl
from jax.experimental.pallas import tpu as pltpu
from jax.experimental.pallas import tpu_sc as plsc

D = 128
DW = 144
NC, NS = 2, 16
CH = 40


def _sc_scatter(xaug, src3, dst3, zrows):
    n = xaug.shape[0]
    rps = (n // NS) // 8 * 8
    tail = n - (NS - 1) * rps
    chunks = src3.shape[1]
    mesh = plsc.VectorSubcoreMesh(core_axis_name="c", subcore_axis_name="s",
                                  num_cores=NC, num_subcores=NS)

    @functools.partial(
        pl.kernel,
        out_type=jax.ShapeDtypeStruct((NC * n, DW), jnp.float32),
        mesh=mesh,
        scratch_types=[
            pltpu.VMEM((chunks, CH), jnp.int32),
            pltpu.VMEM((chunks, CH), jnp.int32),
            pltpu.VMEM((CH, DW), jnp.float32),
            pltpu.VMEM((CH, DW), jnp.float32),
            pltpu.VMEM_SHARED((n, DW), jnp.float32),
            pltpu.SemaphoreType.DMA,
            pltpu.SemaphoreType.DMA,
        ],
        compiler_params=pltpu.CompilerParams(use_tc_tiling_on_sc=False),
    )
    def k(xaug_hbm, src_hbm, dst_hbm, z_hbm, out_hbm, src_v, dst_v, rows0,
          rows1, s_sh, sem0, sem1):
        c = lax.axis_index("c")
        s = lax.axis_index("s")
        wid = c * NS + s
        pltpu.sync_copy(src_hbm.at[wid], src_v)
        pltpu.sync_copy(dst_hbm.at[wid], dst_v)
        @pl.when(s < NS - 1)
        def _z0():
            pltpu.sync_copy(z_hbm.at[pl.ds(0, rps)],
                            s_sh.at[pl.ds(s * rps, rps)])

        @pl.when(s == NS - 1)
        def _z1():
            pltpu.sync_copy(z_hbm, s_sh.at[pl.ds((NS - 1) * rps, tail)])

        plsc.subcore_barrier()

        pltpu.async_copy(xaug_hbm.at[src_v.at[0]], rows0, sem0)
        pltpu.async_copy(xaug_hbm.at[src_v.at[1]], rows1, sem1)

        def pair(j, carry):
            i = 2 * j
            pltpu.make_async_copy(xaug_hbm.at[src_v.at[i]], rows0, sem0).wait()
            pltpu.sync_copy(rows0, s_sh.at[dst_v.at[i]], add=True)
            pltpu.async_copy(xaug_hbm.at[src_v.at[i + 2]], rows0, sem0)
            pltpu.make_async_copy(xaug_hbm.at[src_v.at[i + 1]], rows1,
                                  sem1).wait()
            pltpu.sync_copy(rows1, s_sh.at[dst_v.at[i + 1]], add=True)
            pltpu.async_copy(xaug_hbm.at[src_v.at[i + 3]], rows1, sem1)
            return carry

        lax.fori_loop(0, chunks // 2 - 1, pair, 0)
        pltpu.make_async_copy(xaug_hbm.at[src_v.at[chunks - 2]], rows0,
                              sem0).wait()
        pltpu.sync_copy(rows0, s_sh.at[dst_v.at[chunks - 2]], add=True)
        pltpu.make_async_copy(xaug_hbm.at[src_v.at[chunks - 1]], rows1,
                              sem1).wait()
        pltpu.sync_copy(rows1, s_sh.at[dst_v.at[chunks - 1]], add=True)
        plsc.subcore_barrier()

        @pl.when(s < NS - 1)
        def _r0():
            pltpu.sync_copy(s_sh.at[pl.ds(s * rps, rps)],
                            out_hbm.at[pl.ds(c * n + s * rps, rps)])

        @pl.when(s == NS - 1)
        def _r1():
            pltpu.sync_copy(s_sh.at[pl.ds((NS - 1) * rps, tail)],
                            out_hbm.at[pl.ds(c * n + (NS - 1) * rps, tail)])

    return k(xaug, src3, dst3, zrows)


def _tc_dense(sacc, x, wlt, bl, w1t, b1r, w2pt, b2p):
    n = x.shape[0]
    br = 1000
    g = n // br

    def body(s0, s1, xb, wlt_r, bl_r, w1t_r, b1_r, w2t_r, b2_r, out, acc):
        i = pl.program_id(0)

        @pl.when(i == 0)
        def _init():
            acc[...] = jnp.zeros_like(acc)

        sa = s0[...] + s1[...]
        feat = sa[:, :D] + xb[...]
        cnt = sa[:, D:D + 1] + 1.0
        agg = feat / cnt
        t = jnp.dot(agg, wlt_r[...], preferred_element_type=jnp.float32) + bl_r[...]
        h = jnp.maximum(
            jnp.dot(t, w1t_r[...], preferred_element_type=jnp.float32) + b1_r[...],
            0.0)
        acc[...] = acc[...] + jnp.sum(h, axis=0, keepdims=True)

        @pl.when(i == g - 1)
        def _fin():
            out[...] = (jnp.dot(acc[...], w2t_r[...],
                                preferred_element_type=jnp.float32)
                        + b2_r[...] * float(n))

    full = lambda i: (0, 0)
    return pl.pallas_call(
        body,
        grid=(g,),
        in_specs=[
            pl.BlockSpec((br, DW), lambda i: (i, 0)),
            pl.BlockSpec((br, DW), lambda i, _g=g: (i + _g, 0)),
            pl.BlockSpec((br, D), lambda i: (i, 0)),
            pl.BlockSpec((D, D), full),
            pl.BlockSpec((1, D), full),
            pl.BlockSpec((D, D), full),
            pl.BlockSpec((1, D), full),
            pl.BlockSpec((D, D), full),
            pl.BlockSpec((1, D), full),
        ],
        out_specs=pl.BlockSpec((1, D), full),
        out_shape=jax.ShapeDtypeStruct((1, D), jnp.float32),
        scratch_shapes=[pltpu.VMEM((1, D), jnp.float32)],
    )(sacc, sacc, x, wlt, bl, w1t, b1r, w2pt, b2p)


def kernel(x, edge_index, W_lin, b_lin, W1, b1, W2, b2):
    n, d = x.shape
    e = edge_index.shape[1]
    out_w = W2.shape[0]
    nw = NC * NS
    src3 = edge_index[0].reshape(nw, e // (nw * CH), CH)
    dst3 = edge_index[1].reshape(nw, e // (nw * CH), CH)
    xaug = jnp.concatenate(
        [x, jnp.ones((n, 1), jnp.float32), jnp.zeros((n, DW - d - 1), jnp.float32)],
        axis=1)
    zrows = jnp.zeros((n - (NS - 1) * ((n // NS) // 8 * 8), DW), jnp.float32)
    sacc = _sc_scatter(xaug, src3, dst3, zrows)
    w2pt = jnp.zeros((D, D), jnp.float32).at[:, :out_w].set(W2.T)
    b2p = jnp.zeros((1, D), jnp.float32).at[0, :out_w].set(b2)
    out_row = _tc_dense(sacc, x, W_lin.T, b_lin.reshape(1, D), W1.T,
                        b1.reshape(1, D), w2pt, b2p)
    return out_row[0, :out_w]


# speedup vs baseline: 14.1665x; 1.1770x over previous
import functools

import jax
import jax.numpy as jnp
from jax import lax
from jax.experimental import pallas as pl
from jax.experimental.pallas import tpu as pltpu
from jax.experimental.pallas import tpu_sc as plsc

D = 128
NC, NS = 2, 16
CH = 40
L = 16


def _sc_scatter(x, src3, dst3, zrows):
    n = x.shape[0]
    rps = (n // NS) // 8 * 8
    tail = n - (NS - 1) * rps
    chunks = src3.shape[1]
    mesh = plsc.VectorSubcoreMesh(core_axis_name="c", subcore_axis_name="s",
                                  num_cores=NC, num_subcores=NS)

    @functools.partial(
        pl.kernel,
        out_type=(jax.ShapeDtypeStruct((NC * n, D), jnp.float32),
                  jax.ShapeDtypeStruct((n // 1000, NC * NS, 1000),
                                       jnp.float32)),
        mesh=mesh,
        scratch_types=[
            pltpu.VMEM((chunks, CH), jnp.int32),
            pltpu.VMEM((chunks, CH), jnp.int32),
            pltpu.VMEM((CH, D), jnp.float32),
            pltpu.VMEM((CH, D), jnp.float32),
            pltpu.VMEM((n,), jnp.float32),
            pltpu.VMEM_SHARED((n, D), jnp.float32),
            pltpu.SemaphoreType.DMA,
            pltpu.SemaphoreType.DMA,
        ],
        compiler_params=pltpu.CompilerParams(use_tc_tiling_on_sc=False,
                                             needs_layout_passes=False),
    )
    def k(x_hbm, src_hbm, dst_hbm, z_hbm, out_hbm, cnt_hbm, src_v, dst_v,
          rows0, rows1, cnt_v, s_sh, sem0, sem1):
        c = lax.axis_index("c")
        s = lax.axis_index("s")
        wid = c * NS + s
        pltpu.sync_copy(src_hbm.at[wid], src_v)
        pltpu.sync_copy(dst_hbm.at[wid], dst_v)
        @pl.when(s < NS - 1)
        def _z0():
            pltpu.sync_copy(z_hbm.at[pl.ds(0, rps)],
                            s_sh.at[pl.ds(s * rps, rps)])

        @pl.when(s == NS - 1)
        def _z1():
            pltpu.sync_copy(z_hbm, s_sh.at[pl.ds((NS - 1) * rps, tail)])

        zero16 = jnp.zeros((L,), jnp.float32)

        def zc(i, carry):
            cnt_v[pl.ds(i * L, L)] = zero16
            return carry

        lax.fori_loop(0, n // L, zc, 0)
        plsc.subcore_barrier()

        one16 = jnp.full((L,), 1.0, jnp.float32)
        rem = CH % L
        tail_mask = lax.iota(jnp.int32, L) >= (L - rem)

        def count(i):
            for kk in range(CH // L):
                idx = dst_v[i, pl.ds(kk * L, L)]
                plsc.addupdate_scatter(cnt_v, [idx], one16)
            if rem:
                idx = dst_v[i, pl.ds(CH - L, L)]
                plsc.addupdate_scatter(cnt_v, [idx], one16, mask=tail_mask)

        pltpu.async_copy(x_hbm.at[src_v.at[0]], rows0, sem0)
        pltpu.async_copy(x_hbm.at[src_v.at[1]], rows1, sem1)

        def pair(j, carry):
            i = 2 * j
            count(i)
            pltpu.make_async_copy(x_hbm.at[src_v.at[i]], rows0, sem0).wait()
            pltpu.sync_copy(rows0, s_sh.at[dst_v.at[i]], add=True)
            pltpu.async_copy(x_hbm.at[src_v.at[i + 2]], rows0, sem0)
            count(i + 1)
            pltpu.make_async_copy(x_hbm.at[src_v.at[i + 1]], rows1,
                                  sem1).wait()
            pltpu.sync_copy(rows1, s_sh.at[dst_v.at[i + 1]], add=True)
            pltpu.async_copy(x_hbm.at[src_v.at[i + 3]], rows1, sem1)
            return carry

        lax.fori_loop(0, chunks // 2 - 1, pair, 0)
        count(chunks - 2)
        pltpu.make_async_copy(x_hbm.at[src_v.at[chunks - 2]], rows0,
                              sem0).wait()
        pltpu.sync_copy(rows0, s_sh.at[dst_v.at[chunks - 2]], add=True)
        count(chunks - 1)
        pltpu.make_async_copy(x_hbm.at[src_v.at[chunks - 1]], rows1,
                              sem1).wait()
        pltpu.sync_copy(rows1, s_sh.at[dst_v.at[chunks - 1]], add=True)
        plsc.subcore_barrier()

        for b in range(n // 1000):
            pltpu.sync_copy(cnt_v.at[pl.ds(b * 1000, 1000)],
                            cnt_hbm.at[b, wid])

        @pl.when(s < NS - 1)
        def _r0():
            pltpu.sync_copy(s_sh.at[pl.ds(s * rps, rps)],
                            out_hbm.at[pl.ds(c * n + s * rps, rps)])

        @pl.when(s == NS - 1)
        def _r1():
            pltpu.sync_copy(s_sh.at[pl.ds((NS - 1) * rps, tail)],
                            out_hbm.at[pl.ds(c * n + (NS - 1) * rps, tail)])

    return k(x, src3, dst3, zrows)


def _tc_dense(sacc, cnt, x, wlt, bl, w1t, b1r, w2pt, b2p):
    n = x.shape[0]
    nw = cnt.shape[1]
    br = 1000
    g = n // br

    def body(s0, s1, cnt_r, xb, wlt_r, bl_r, w1t_r, b1_r, w2t_r, b2_r, out,
             acc):
        i = pl.program_id(0)

        @pl.when(i == 0)
        def _init():
            acc[...] = jnp.zeros_like(acc)

        ones_col = jnp.ones((nw, 1), jnp.float32)
        cnt_col = lax.dot_general(cnt_r[0], ones_col, (((0,), (0,)), ((), ())),
                                  preferred_element_type=jnp.float32) + 1.0
        feat = s0[...] + s1[...] + xb[...]
        agg = feat / cnt_col
        t = jnp.dot(agg, wlt_r[...], preferred_element_type=jnp.float32) + bl_r[...]
        h = jnp.maximum(
            jnp.dot(t, w1t_r[...], preferred_element_type=jnp.float32) + b1_r[...],
            0.0)
        acc[...] = acc[...] + jnp.sum(h, axis=0, keepdims=True)

        @pl.when(i == g - 1)
        def _fin():
            out[...] = (jnp.dot(acc[...], w2t_r[...],
                                preferred_element_type=jnp.float32)
                        + b2_r[...] * float(n))

    full = lambda i: (0, 0)
    return pl.pallas_call(
        body,
        grid=(g,),
        in_specs=[
            pl.BlockSpec((br, D), lambda i: (i, 0)),
            pl.BlockSpec((br, D), lambda i, _g=g: (i + _g, 0)),
            pl.BlockSpec((1, nw, br), lambda i: (i, 0, 0)),
            pl.BlockSpec((br, D), lambda i: (i, 0)),
            pl.BlockSpec((D, D), full),
            pl.BlockSpec((1, D), full),
            pl.BlockSpec((D, D), full),
            pl.BlockSpec((1, D), full),
            pl.BlockSpec((D, D), full),
            pl.BlockSpec((1, D), full),
        ],
        out_specs=pl.BlockSpec((1, D), full),
        out_shape=jax.ShapeDtypeStruct((1, D), jnp.float32),
        scratch_shapes=[pltpu.VMEM((1, D), jnp.float32)],
    )(sacc, sacc, cnt, x, wlt, bl, w1t, b1r, w2pt, b2p)


def kernel(x, edge_index, W_lin, b_lin, W1, b1, W2, b2):
    n, d = x.shape
    e = edge_index.shape[1]
    out_w = W2.shape[0]
    nw = NC * NS
    src3 = edge_index[0].reshape(nw, e // (nw * CH), CH)
    dst3 = edge_index[1].reshape(nw, e // (nw * CH), CH)
    zrows = jnp.zeros((n - (NS - 1) * ((n // NS) // 8 * 8), D), jnp.float32)
    sacc, cnt = _sc_scatter(x, src3, dst3, zrows)
    w2pt = jnp.zeros((D, D), jnp.float32).at[:, :out_w].set(W2.T)
    b2p = jnp.zeros((1, D), jnp.float32).at[0, :out_w].set(b2)
    out_row = _tc_dense(sacc, cnt, x, W_lin.T, b_lin.reshape(1, D), W1.T,
                        b1.reshape(1, D), w2pt, b2p)
    return out_row[0, :out_w]


# speedup vs baseline: 17.4832x; 1.2341x over previous
import functools

import jax
import jax.numpy as jnp
from jax import lax
from jax.experimental import pallas as pl
from jax.experimental.pallas import tpu as pltpu
from jax.experimental.pallas import tpu_sc as plsc

D = 128
NC, NS = 2, 16
CH = 80
L = 16


def _sc_scatter(x, src3, dst3, zrows):
    n = x.shape[0]
    rps = (n // NS) // 8 * 8
    tail = n - (NS - 1) * rps
    chunks = src3.shape[1]
    mesh = plsc.VectorSubcoreMesh(core_axis_name="c", subcore_axis_name="s",
                                  num_cores=NC, num_subcores=NS)

    @functools.partial(
        pl.kernel,
        out_type=(jax.ShapeDtypeStruct((NC * n, D), jnp.float32),
                  jax.ShapeDtypeStruct((n // 1000, NC * NS, 1000),
                                       jnp.float32)),
        mesh=mesh,
        scratch_types=[
            pltpu.VMEM((chunks, CH), jnp.int32),
            pltpu.VMEM((chunks, CH), jnp.int32),
            pltpu.VMEM((CH, D), jnp.float32),
            pltpu.VMEM((CH, D), jnp.float32),
            pltpu.VMEM((n,), jnp.float32),
            pltpu.VMEM_SHARED((n, D), jnp.float32),
            pltpu.SemaphoreType.DMA,
            pltpu.SemaphoreType.DMA,
        ],
        compiler_params=pltpu.CompilerParams(use_tc_tiling_on_sc=False,
                                             needs_layout_passes=False),
    )
    def k(x_hbm, src_hbm, dst_hbm, z_hbm, out_hbm, cnt_hbm, src_v, dst_v,
          rows0, rows1, cnt_v, s_sh, sem0, sem1):
        c = lax.axis_index("c")
        s = lax.axis_index("s")
        wid = c * NS + s
        pltpu.sync_copy(src_hbm.at[wid], src_v)
        pltpu.sync_copy(dst_hbm.at[wid], dst_v)
        @pl.when(s < NS - 1)
        def _z0():
            pltpu.sync_copy(z_hbm.at[pl.ds(0, rps)],
                            s_sh.at[pl.ds(s * rps, rps)])

        @pl.when(s == NS - 1)
        def _z1():
            pltpu.sync_copy(z_hbm, s_sh.at[pl.ds((NS - 1) * rps, tail)])

        zero16 = jnp.zeros((L,), jnp.float32)

        def zc(i, carry):
            cnt_v[pl.ds(i * L, L)] = zero16
            return carry

        lax.fori_loop(0, n // L, zc, 0)
        plsc.subcore_barrier()

        one16 = jnp.full((L,), 1.0, jnp.float32)
        rem = CH % L
        tail_mask = lax.iota(jnp.int32, L) >= (L - rem)

        def count(i):
            for kk in range(CH // L):
                idx = dst_v[i, pl.ds(kk * L, L)]
                plsc.addupdate_scatter(cnt_v, [idx], one16)
            if rem:
                idx = dst_v[i, pl.ds(CH - L, L)]
                plsc.addupdate_scatter(cnt_v, [idx], one16, mask=tail_mask)

        pltpu.async_copy(x_hbm.at[src_v.at[0]], rows0, sem0)
        pltpu.async_copy(x_hbm.at[src_v.at[1]], rows1, sem1)

        def retire(i, rows, sem):
            count(i)
            pltpu.make_async_copy(x_hbm.at[src_v.at[i]], rows, sem).wait()
            pltpu.sync_copy(rows, s_sh.at[dst_v.at[i]], add=True)

        def pair(j, carry):
            i = 2 * j
            retire(i, rows0, sem0)
            pltpu.async_copy(x_hbm.at[src_v.at[i + 2]], rows0, sem0)
            retire(i + 1, rows1, sem1)
            pltpu.async_copy(x_hbm.at[src_v.at[i + 3]], rows1, sem1)
            return carry

        lax.fori_loop(0, (chunks - 3) // 2, pair, 0)
        retire(chunks - 3, rows0, sem0)
        pltpu.async_copy(x_hbm.at[src_v.at[chunks - 1]], rows0, sem0)
        retire(chunks - 2, rows1, sem1)
        retire(chunks - 1, rows0, sem0)
        plsc.subcore_barrier()

        for b in range(n // 1000):
            pltpu.sync_copy(cnt_v.at[pl.ds(b * 1000, 1000)],
                            cnt_hbm.at[b, wid])

        @pl.when(s < NS - 1)
        def _r0():
            pltpu.sync_copy(s_sh.at[pl.ds(s * rps, rps)],
                            out_hbm.at[pl.ds(c * n + s * rps, rps)])

        @pl.when(s == NS - 1)
        def _r1():
            pltpu.sync_copy(s_sh.at[pl.ds((NS - 1) * rps, tail)],
                            out_hbm.at[pl.ds(c * n + (NS - 1) * rps, tail)])

    return k(x, src3, dst3, zrows)


def _tc_dense(sacc, cnt, x, wlt, bl, w1t, b1r, w2pt, b2p):
    n = x.shape[0]
    nw = cnt.shape[1]
    br = 1000
    g = n // br

    def body(s0, s1, cnt_r, xb, wlt_r, bl_r, w1t_r, b1_r, w2t_r, b2_r, out,
             acc):
        i = pl.program_id(0)

        @pl.when(i == 0)
        def _init():
            acc[...] = jnp.zeros_like(acc)

        ones_col = jnp.ones((nw, 1), jnp.float32)
        cnt_col = lax.dot_general(cnt_r[0], ones_col, (((0,), (0,)), ((), ())),
                                  preferred_element_type=jnp.float32) + 1.0
        feat = s0[...] + s1[...] + xb[...]
        agg = feat / cnt_col
        t = jnp.dot(agg, wlt_r[...], preferred_element_type=jnp.float32) + bl_r[...]
        h = jnp.maximum(
            jnp.dot(t, w1t_r[...], preferred_element_type=jnp.float32) + b1_r[...],
            0.0)
        acc[...] = acc[...] + jnp.sum(h, axis=0, keepdims=True)

        @pl.when(i == g - 1)
        def _fin():
            out[...] = (jnp.dot(acc[...], w2t_r[...],
                                preferred_element_type=jnp.float32)
                        + b2_r[...] * float(n))

    full = lambda i: (0, 0)
    return pl.pallas_call(
        body,
        grid=(g,),
        in_specs=[
            pl.BlockSpec((br, D), lambda i: (i, 0)),
            pl.BlockSpec((br, D), lambda i, _g=g: (i + _g, 0)),
            pl.BlockSpec((1, nw, br), lambda i: (i, 0, 0)),
            pl.BlockSpec((br, D), lambda i: (i, 0)),
            pl.BlockSpec((D, D), full),
            pl.BlockSpec((1, D), full),
            pl.BlockSpec((D, D), full),
            pl.BlockSpec((1, D), full),
            pl.BlockSpec((D, D), full),
            pl.BlockSpec((1, D), full),
        ],
        out_specs=pl.BlockSpec((1, D), full),
        out_shape=jax.ShapeDtypeStruct((1, D), jnp.float32),
        scratch_shapes=[pltpu.VMEM((1, D), jnp.float32)],
    )(sacc, sacc, cnt, x, wlt, bl, w1t, b1r, w2pt, b2p)


def kernel(x, edge_index, W_lin, b_lin, W1, b1, W2, b2):
    n, d = x.shape
    e = edge_index.shape[1]
    out_w = W2.shape[0]
    nw = NC * NS
    src3 = edge_index[0].reshape(nw, e // (nw * CH), CH)
    dst3 = edge_index[1].reshape(nw, e // (nw * CH), CH)
    zrows = jnp.zeros((n - (NS - 1) * ((n // NS) // 8 * 8), D), jnp.float32)
    sacc, cnt = _sc_scatter(x, src3, dst3, zrows)
    w2pt = jnp.zeros((D, D), jnp.float32).at[:, :out_w].set(W2.T)
    b2p = jnp.zeros((1, D), jnp.float32).at[0, :out_w].set(b2)
    out_row = _tc_dense(sacc, cnt, x, W_lin.T, b_lin.reshape(1, D), W1.T,
                        b1.reshape(1, D), w2pt, b2p)
    return out_row[0, :out_w]
